# baseline (speedup 1.0000x reference)
import functools

import jax
import jax.numpy as jnp
from jax import lax
from jax.experimental import pallas as pl
from jax.experimental.pallas import tpu as pltpu
from jax.experimental.pallas import tpu_sc as plsc

B = 1024
IN = 1024
OUT = 1024
H = 16
P = 8
DK = OUT // H
FH = 32
FW = 32
C = DK
HW = FH * FW
L = H * P
N = B * H

_NW = 32
_BPW = B // _NW


def _transpose_body(fm_ref, out_ref):
    fm = fm_ref[...].reshape(fm_ref.shape[0], C, HW)
    t = jnp.swapaxes(fm, 1, 2)
    out_ref[...] = jnp.concatenate([t, jnp.zeros_like(t)], axis=2)


def _build_table(fm):
    blk = 4
    return pl.pallas_call(
        _transpose_body,
        grid=(B // blk,),
        in_specs=[pl.BlockSpec((blk, C, FH, FW), lambda i: (i, 0, 0, 0))],
        out_specs=pl.BlockSpec((blk, HW, 2 * C), lambda i: (i, 0, 0)),
        out_shape=jax.ShapeDtypeStruct((B, HW, 2 * C), jnp.float32),
    )(fm)


def _proj_body(x_ref, wq_ref, bq_ref, woff_ref, boff_ref, wpv_ref, bpv_ref,
               q_ref, idx_ref0, idx_ref1, idx_ref2, idx_ref3,
               w_ref0, w_ref1, w_ref2, w_ref3, pv_ref):
    i = pl.program_id(0)
    x = x_ref[...]
    q_ref[...] = jnp.dot(x, wq_ref[...],
                         preferred_element_type=jnp.float32) + bq_ref[...]
    off = jnp.dot(x, woff_ref[...],
                  preferred_element_type=jnp.float32) + boff_ref[...]
    pv = jnp.dot(x, wpv_ref[...],
                 preferred_element_type=jnp.float32) + bpv_ref[...]
    pv_ref[...] = pv
    bm = x.shape[0]
    gx = off[:, :L] + pv[:, 0:1]
    gy = off[:, L:] + pv[:, 1:2]
    ix = ((gx + 1.0) * FW - 1.0) * 0.5
    iy = ((gy + 1.0) * FH - 1.0) * 0.5
    ix0 = jnp.floor(ix)
    iy0 = jnp.floor(iy)
    wx1 = ix - ix0
    wx0 = 1.0 - wx1
    wy1 = iy - iy0
    wy0 = 1.0 - wy1
    rowg = i * bm + lax.broadcasted_iota(jnp.int32, (bm, L), 0)
    lane = lax.broadcasted_iota(jnp.int32, (bm, L), 1)
    img = (rowg * H + lane // P) % B
    idx_refs = (idx_ref0, idx_ref1, idx_ref2, idx_ref3)
    w_refs = (w_ref0, w_ref1, w_ref2, w_ref3)
    for k, (dy, dx, wx, wy) in enumerate(
            ((0, 0, wx0, wy0), (0, 1, wx1, wy0),
             (1, 0, wx0, wy1), (1, 1, wx1, wy1))):
        xf = ix0 + dx
        yf = iy0 + dy
        valid = ((xf >= 0) & (xf <= FW - 1) & (yf >= 0) & (yf <= FH - 1))
        xi = jnp.clip(xf, 0, FW - 1).astype(jnp.int32)
        yi = jnp.clip(yf, 0, FH - 1).astype(jnp.int32)
        idx_refs[k][...] = img * HW + yi * FW + xi
        w_refs[k][...] = wx * wy * valid.astype(jnp.float32)


def _projections(x, Wq, bq2, Woffp, boffp2, Wpv, bpv2):
    bm = 256
    grid = (B // bm,)
    return pl.pallas_call(
        _proj_body,
        grid=grid,
        in_specs=[
            pl.BlockSpec((bm, IN), lambda i: (i, 0)),
            pl.BlockSpec((IN, OUT), lambda i: (0, 0)),
            pl.BlockSpec((1, OUT), lambda i: (0, 0)),
            pl.BlockSpec((IN, 2 * L), lambda i: (0, 0)),
            pl.BlockSpec((1, 2 * L), lambda i: (0, 0)),
            pl.BlockSpec((IN, 4), lambda i: (0, 0)),
            pl.BlockSpec((1, 4), lambda i: (0, 0)),
        ],
        out_specs=[pl.BlockSpec((bm, OUT), lambda i: (i, 0))]
        + [pl.BlockSpec((bm, L), lambda i: (i, 0))] * 8
        + [pl.BlockSpec((bm, 4), lambda i: (i, 0))],
        out_shape=[jax.ShapeDtypeStruct((B, OUT), jnp.float32)]
        + [jax.ShapeDtypeStruct((B, L), jnp.int32)] * 4
        + [jax.ShapeDtypeStruct((B, L), jnp.float32)] * 4
        + [jax.ShapeDtypeStruct((B, 4), jnp.float32)],
    )(x, Wq, bq2, Woffp, boffp2, Wpv, bpv2)


def _sc_gather_combine(table, idxs, ws):
    mesh = plsc.VectorSubcoreMesh(core_axis_name="c", subcore_axis_name="s")

    @functools.partial(
        pl.kernel,
        mesh=mesh,
        out_type=jax.ShapeDtypeStruct((N, P * DK), jnp.float32),
        scratch_types=[
            pltpu.VMEM((4, _BPW, L), jnp.int32),
            pltpu.VMEM((4, _BPW, L), jnp.float32),
            pltpu.VMEM((4 * L, 2 * C), jnp.float32),
            pltpu.VMEM((H, P * DK), jnp.float32),
            pltpu.SemaphoreType.DMA,
        ],
    )
    def k(table_hbm, i0, i1, i2, i3, w0, w1, w2, w3, out_hbm,
          idx_v, w_v, rows_v, out_v, sem):
        wid = lax.axis_index("s") * 2 + lax.axis_index("c")
        base = wid * _BPW
        for kk, r in enumerate((i0, i1, i2, i3)):
            pltpu.sync_copy(r.at[pl.ds(base, _BPW)], idx_v.at[kk])
        for kk, r in enumerate((w0, w1, w2, w3)):
            pltpu.sync_copy(r.at[pl.ds(base, _BPW)], w_v.at[kk])

        def body(t, _):
            cps = [
                pltpu.async_copy(table_hbm.at[idx_v.at[kk, t]],
                                 rows_v.at[pl.ds(kk * L, L)], sem)
                for kk in range(4)
            ]
            for cp in cps:
                cp.wait()

            def inner(g, _):
                wv = [w_v[kk, t, pl.ds(g * 16, 16)] for kk in range(4)]
                for j in range(16):
                    l = g * 16 + j
                    ro = g * 2 + j // P
                    co = (j % P) * DK
                    for c4 in range(C // 16):
                        s = pl.ds(c4 * 16, 16)
                        acc = wv[0][j] * rows_v[l, s]
                        acc = acc + wv[1][j] * rows_v[L + l, s]
                        acc = acc + wv[2][j] * rows_v[2 * L + l, s]
                        acc = acc + wv[3][j] * rows_v[3 * L + l, s]
                        out_v[ro, pl.ds(co + c4 * 16, 16)] = acc
                return 0

            lax.fori_loop(0, L // 16, inner, 0)
            pltpu.sync_copy(out_v, out_hbm.at[pl.ds((base + t) * H, H)])
            return 0

        lax.fori_loop(0, _BPW, body, 0)

    return k(table, *idxs, *ws)


def _attn_body(q_ref, s_ref, o_ref):
    q = q_ref[...]
    s = s_ref[...]
    dots = [jnp.sum(q * s[:, p * DK:(p + 1) * DK], axis=1, keepdims=True)
            for p in range(P)]
    scores = jnp.concatenate(dots, axis=1) * (DK ** -0.5)
    m = jnp.max(scores, axis=1, keepdims=True)
    e = jnp.exp(scores - m)
    a = e / jnp.sum(e, axis=1, keepdims=True)
    o = a[:, 0:1] * s[:, :DK]
    for p in range(1, P):
        o = o + a[:, p:p + 1] * s[:, p * DK:(p + 1) * DK]
    o_ref[...] = o


def _attention(q2, s2):
    br = 2048
    return pl.pallas_call(
        _attn_body,
        grid=(N // br,),
        in_specs=[
            pl.BlockSpec((br, DK), lambda i: (i, 0)),
            pl.BlockSpec((br, P * DK), lambda i: (i, 0)),
        ],
        out_specs=pl.BlockSpec((br, DK), lambda i: (i, 0)),
        out_shape=jax.ShapeDtypeStruct((N, DK), jnp.float32),
    )(q2, s2)


def _mm_body(a_ref, w_ref, b_ref, o_ref):
    o_ref[...] = jnp.dot(a_ref[...], w_ref[...],
                         preferred_element_type=jnp.float32) + b_ref[...]


def _out_matmul(a, Wout, bout2):
    bm = 128
    return pl.pallas_call(
        _mm_body,
        grid=(B // bm,),
        in_specs=[
            pl.BlockSpec((bm, OUT), lambda i: (i, 0)),
            pl.BlockSpec((OUT, OUT), lambda i: (0, 0)),
            pl.BlockSpec((1, OUT), lambda i: (0, 0)),
        ],
        out_specs=pl.BlockSpec((bm, OUT), lambda i: (i, 0)),
        out_shape=jax.ShapeDtypeStruct((B, OUT), jnp.float32),
    )(a, Wout, bout2)


@jax.jit
def kernel(x, feature_map, Wq, bq, Woff, boff, Wpos, bpos, Wvel, bvel,
           Wout, bout):
    Woffp = Woff.reshape(IN, L, 2).transpose(0, 2, 1).reshape(IN, 2 * L)
    boffp = boff.reshape(L, 2).transpose(1, 0).reshape(2 * L)
    Wpv = jnp.concatenate([Wpos, Wvel], axis=1)
    bpv = jnp.concatenate([bpos, bvel])

    table = _build_table(feature_map).reshape(B * HW, 2 * C)
    (q, idx0, idx1, idx2, idx3, w0, w1, w2, w3, pv) = _projections(
        x, Wq, bq.reshape(1, OUT), Woffp, boffp.reshape(1, 2 * L),
        Wpv, bpv.reshape(1, 4))
    samp = _sc_gather_combine(table, (idx0, idx1, idx2, idx3),
                              (w0, w1, w2, w3))
    aout = _attention(q.reshape(N, DK), samp)
    out = _out_matmul(aout.reshape(B, OUT), Wout, bout.reshape(1, OUT))
    return out, pv[:, 0:2], pv[:, 2:4]


# speedup vs baseline: 1.8777x; 1.8777x over previous
import functools

import jax
import jax.numpy as jnp
from jax import lax
from jax.experimental import pallas as pl
from jax.experimental.pallas import tpu as pltpu
from jax.experimental.pallas import tpu_sc as plsc

B = 1024
IN = 1024
OUT = 1024
H = 16
P = 8
DK = OUT // H
FH = 32
FW = 32
C = DK
HW = FH * FW
L = H * P
N = B * H

_NW = 32
_BPW = B // _NW


def _transpose_body(fm_ref, out_ref):
    t = jnp.swapaxes(fm_ref[...], 1, 2)
    out_ref[...] = jnp.concatenate([t, jnp.zeros_like(t)], axis=2)


def _build_table(fm3):
    blk = 8
    return pl.pallas_call(
        _transpose_body,
        grid=(B // blk,),
        in_specs=[pl.BlockSpec((blk, C, HW), lambda i: (i, 0, 0))],
        out_specs=pl.BlockSpec((blk, HW, 2 * C), lambda i: (i, 0, 0)),
        out_shape=jax.ShapeDtypeStruct((B, HW, 2 * C), jnp.float32),
    )(fm3)


def _proj_body(x_ref, wq_ref, bq_ref, woff_ref, boff_ref, wpv_ref, bpv_ref,
               q_ref, idx_ref0, idx_ref1, idx_ref2, idx_ref3,
               w_ref0, w_ref1, w_ref2, w_ref3, pv_ref):
    i = pl.program_id(0)
    x = x_ref[...]
    q_ref[...] = jnp.dot(x, wq_ref[...],
                         preferred_element_type=jnp.float32) + bq_ref[...]
    off = jnp.dot(x, woff_ref[...],
                  preferred_element_type=jnp.float32) + boff_ref[...]
    pv = jnp.dot(x, wpv_ref[...],
                 preferred_element_type=jnp.float32) + bpv_ref[...]
    pv_ref[...] = pv
    bm = x.shape[0]
    gx = off[:, :L] + pv[:, 0:1]
    gy = off[:, L:] + pv[:, 1:2]
    ix = ((gx + 1.0) * FW - 1.0) * 0.5
    iy = ((gy + 1.0) * FH - 1.0) * 0.5
    ix0 = jnp.floor(ix)
    iy0 = jnp.floor(iy)
    wx1 = ix - ix0
    wx0 = 1.0 - wx1
    wy1 = iy - iy0
    wy0 = 1.0 - wy1
    rowg = i * bm + lax.broadcasted_iota(jnp.int32, (bm, L), 0)
    lane = lax.broadcasted_iota(jnp.int32, (bm, L), 1)
    img = (rowg * H + lane // P) % B
    idx_refs = (idx_ref0, idx_ref1, idx_ref2, idx_ref3)
    w_refs = (w_ref0, w_ref1, w_ref2, w_ref3)
    for k, (dy, dx, wx, wy) in enumerate(
            ((0, 0, wx0, wy0), (0, 1, wx1, wy0),
             (1, 0, wx0, wy1), (1, 1, wx1, wy1))):
        xf = ix0 + dx
        yf = iy0 + dy
        valid = ((xf >= 0) & (xf <= FW - 1) & (yf >= 0) & (yf <= FH - 1))
        xi = jnp.clip(xf, 0, FW - 1).astype(jnp.int32)
        yi = jnp.clip(yf, 0, FH - 1).astype(jnp.int32)
        idx_refs[k][...] = img * HW + yi * FW + xi
        w_refs[k][...] = wx * wy * valid.astype(jnp.float32)


def _projections(x, Wq, bq2, Woffp, boffp2, Wpv, bpv2):
    bm = 256
    grid = (B // bm,)
    return pl.pallas_call(
        _proj_body,
        grid=grid,
        in_specs=[
            pl.BlockSpec((bm, IN), lambda i: (i, 0)),
            pl.BlockSpec((IN, OUT), lambda i: (0, 0)),
            pl.BlockSpec((1, OUT), lambda i: (0, 0)),
            pl.BlockSpec((IN, 2 * L), lambda i: (0, 0)),
            pl.BlockSpec((1, 2 * L), lambda i: (0, 0)),
            pl.BlockSpec((IN, 4), lambda i: (0, 0)),
            pl.BlockSpec((1, 4), lambda i: (0, 0)),
        ],
        out_specs=[pl.BlockSpec((bm, OUT), lambda i: (i, 0))]
        + [pl.BlockSpec((bm, L), lambda i: (i, 0))] * 8
        + [pl.BlockSpec((bm, 4), lambda i: (i, 0))],
        out_shape=[jax.ShapeDtypeStruct((B, OUT), jnp.float32)]
        + [jax.ShapeDtypeStruct((B, L), jnp.int32)] * 4
        + [jax.ShapeDtypeStruct((B, L), jnp.float32)] * 4
        + [jax.ShapeDtypeStruct((B, 4), jnp.float32)],
    )(x, Wq, bq2, Woffp, boffp2, Wpv, bpv2)


def _sc_gather_combine(table, idxs, ws):
    mesh = plsc.VectorSubcoreMesh(core_axis_name="c", subcore_axis_name="s")

    @functools.partial(
        pl.kernel,
        mesh=mesh,
        out_type=jax.ShapeDtypeStruct((N, P * DK), jnp.float32),
        scratch_types=[
            pltpu.VMEM((4, _BPW, L), jnp.int32),
            pltpu.VMEM((4, _BPW, L), jnp.float32),
            pltpu.VMEM((4 * L, 2 * C), jnp.float32),
            pltpu.VMEM((H, P * DK), jnp.float32),
            pltpu.SemaphoreType.DMA,
            pltpu.SemaphoreType.DMA,
        ],
    )
    def k(table_hbm, i0, i1, i2, i3, w0, w1, w2, w3, out_hbm,
          idx_v, w_v, rows_v, out_v, semA, semB):
        wid = lax.axis_index("s") * 2 + lax.axis_index("c")
        base = wid * _BPW
        for kk, r in enumerate((i0, i1, i2, i3)):
            pltpu.sync_copy(r.at[pl.ds(base, _BPW)], idx_v.at[kk])
        for kk, r in enumerate((w0, w1, w2, w3)):
            pltpu.sync_copy(r.at[pl.ds(base, _BPW)], w_v.at[kk])

        def body(t, _):
            cpsA = [
                pltpu.async_copy(table_hbm.at[idx_v.at[kk, t]],
                                 rows_v.at[pl.ds(kk * L, L)], semA)
                for kk in range(2)
            ]
            cpsB = [
                pltpu.async_copy(table_hbm.at[idx_v.at[kk, t]],
                                 rows_v.at[pl.ds(kk * L, L)], semB)
                for kk in range(2, 4)
            ]
            for cp in cpsA:
                cp.wait()

            def innerA(g, _):
                wv = [w_v[kk, t, pl.ds(g * 16, 16)] for kk in range(2)]
                for j in range(16):
                    l = g * 16 + j
                    ro = g * 2 + j // P
                    co = (j % P) * DK
                    for c4 in range(C // 16):
                        s = pl.ds(c4 * 16, 16)
                        acc = wv[0][j] * rows_v[l, s]
                        acc = acc + wv[1][j] * rows_v[L + l, s]
                        out_v[ro, pl.ds(co + c4 * 16, 16)] = acc
                return 0

            lax.fori_loop(0, L // 16, innerA, 0)
            for cp in cpsB:
                cp.wait()

            def innerB(g, _):
                wv = [w_v[kk, t, pl.ds(g * 16, 16)] for kk in range(2, 4)]
                for j in range(16):
                    l = g * 16 + j
                    ro = g * 2 + j // P
                    co = (j % P) * DK
                    for c4 in range(C // 16):
                        s = pl.ds(c4 * 16, 16)
                        acc = out_v[ro, pl.ds(co + c4 * 16, 16)]
                        acc = acc + wv[0][j] * rows_v[2 * L + l, s]
                        acc = acc + wv[1][j] * rows_v[3 * L + l, s]
                        out_v[ro, pl.ds(co + c4 * 16, 16)] = acc
                return 0

            lax.fori_loop(0, L // 16, innerB, 0)
            pltpu.sync_copy(out_v, out_hbm.at[pl.ds((base + t) * H, H)])
            return 0

        lax.fori_loop(0, _BPW, body, 0)

    return k(table, *idxs, *ws)


def _attn_body(q_ref, s_ref, o_ref):
    q = q_ref[...]
    s = s_ref[...]
    dots = [jnp.sum(q * s[:, p * DK:(p + 1) * DK], axis=1, keepdims=True)
            for p in range(P)]
    scores = jnp.concatenate(dots, axis=1) * (DK ** -0.5)
    m = jnp.max(scores, axis=1, keepdims=True)
    e = jnp.exp(scores - m)
    a = e / jnp.sum(e, axis=1, keepdims=True)
    o = a[:, 0:1] * s[:, :DK]
    for p in range(1, P):
        o = o + a[:, p:p + 1] * s[:, p * DK:(p + 1) * DK]
    o_ref[...] = o


def _attention(q2, s2):
    br = 2048
    return pl.pallas_call(
        _attn_body,
        grid=(N // br,),
        in_specs=[
            pl.BlockSpec((br, DK), lambda i: (i, 0)),
            pl.BlockSpec((br, P * DK), lambda i: (i, 0)),
        ],
        out_specs=pl.BlockSpec((br, DK), lambda i: (i, 0)),
        out_shape=jax.ShapeDtypeStruct((N, DK), jnp.float32),
    )(q2, s2)


def _mm_body(a_ref, w_ref, b_ref, o_ref):
    o_ref[...] = jnp.dot(a_ref[...], w_ref[...],
                         preferred_element_type=jnp.float32) + b_ref[...]


def _out_matmul(a, Wout, bout2):
    bm = 128
    return pl.pallas_call(
        _mm_body,
        grid=(B // bm,),
        in_specs=[
            pl.BlockSpec((bm, OUT), lambda i: (i, 0)),
            pl.BlockSpec((OUT, OUT), lambda i: (0, 0)),
            pl.BlockSpec((1, OUT), lambda i: (0, 0)),
        ],
        out_specs=pl.BlockSpec((bm, OUT), lambda i: (i, 0)),
        out_shape=jax.ShapeDtypeStruct((B, OUT), jnp.float32),
    )(a, Wout, bout2)


@jax.jit
def kernel(x, feature_map, Wq, bq, Woff, boff, Wpos, bpos, Wvel, bvel,
           Wout, bout):
    Woffp = Woff.reshape(IN, L, 2).transpose(0, 2, 1).reshape(IN, 2 * L)
    boffp = boff.reshape(L, 2).transpose(1, 0).reshape(2 * L)
    Wpv = jnp.concatenate([Wpos, Wvel], axis=1)
    bpv = jnp.concatenate([bpos, bvel])

    table = _build_table(feature_map.reshape(B, C, HW)).reshape(B * HW, 2 * C)
    (q, idx0, idx1, idx2, idx3, w0, w1, w2, w3, pv) = _projections(
        x, Wq, bq.reshape(1, OUT), Woffp, boffp.reshape(1, 2 * L),
        Wpv, bpv.reshape(1, 4))
    samp = _sc_gather_combine(table, (idx0, idx1, idx2, idx3),
                              (w0, w1, w2, w3))
    aout = _attention(q.reshape(N, DK), samp)
    out = _out_matmul(aout.reshape(B, OUT), Wout, bout.reshape(1, OUT))
    return out, pv[:, 0:2], pv[:, 2:4]


# speedup vs baseline: 1.8838x; 1.0033x over previous
import functools

import jax
import jax.numpy as jnp
from jax import lax
from jax.experimental import pallas as pl
from jax.experimental.pallas import tpu as pltpu
from jax.experimental.pallas import tpu_sc as plsc

B = 1024
IN = 1024
OUT = 1024
H = 16
P = 8
DK = OUT // H
FH = 32
FW = 32
C = DK
HW = FH * FW
L = H * P
N = B * H

_NW = 32
_BPW = B // _NW


def _transpose_body(fm_ref, out_ref):
    out_ref[:, :, 0:C] = jnp.swapaxes(fm_ref[...], 1, 2)


def _build_table(fm3):
    blk = 8
    return pl.pallas_call(
        _transpose_body,
        grid=(B // blk,),
        in_specs=[pl.BlockSpec((blk, C, HW), lambda i: (i, 0, 0))],
        out_specs=pl.BlockSpec((blk, HW, 2 * C), lambda i: (i, 0, 0)),
        out_shape=jax.ShapeDtypeStruct((B, HW, 2 * C), jnp.float32),
    )(fm3)


def _proj_body(x_ref, wq_ref, bq_ref, woff_ref, boff_ref, wpv_ref, bpv_ref,
               q_ref, idx_ref0, idx_ref1, idx_ref2, idx_ref3,
               w_ref0, w_ref1, w_ref2, w_ref3, pv_ref):
    i = pl.program_id(0)
    x = x_ref[...]
    q_ref[...] = jnp.dot(x, wq_ref[...],
                         preferred_element_type=jnp.float32) + bq_ref[...]
    off = jnp.dot(x, woff_ref[...],
                  preferred_element_type=jnp.float32) + boff_ref[...]
    pv = jnp.dot(x, wpv_ref[...],
                 preferred_element_type=jnp.float32) + bpv_ref[...]
    pv_ref[...] = pv
    bm = x.shape[0]
    gx = off[:, :L] + pv[:, 0:1]
    gy = off[:, L:] + pv[:, 1:2]
    ix = ((gx + 1.0) * FW - 1.0) * 0.5
    iy = ((gy + 1.0) * FH - 1.0) * 0.5
    ix0 = jnp.floor(ix)
    iy0 = jnp.floor(iy)
    wx1 = ix - ix0
    wx0 = 1.0 - wx1
    wy1 = iy - iy0
    wy0 = 1.0 - wy1
    rowg = i * bm + lax.broadcasted_iota(jnp.int32, (bm, L), 0)
    lane = lax.broadcasted_iota(jnp.int32, (bm, L), 1)
    img = (rowg * H + lane // P) % B
    idx_refs = (idx_ref0, idx_ref1, idx_ref2, idx_ref3)
    w_refs = (w_ref0, w_ref1, w_ref2, w_ref3)
    for k, (dy, dx, wx, wy) in enumerate(
            ((0, 0, wx0, wy0), (0, 1, wx1, wy0),
             (1, 0, wx0, wy1), (1, 1, wx1, wy1))):
        xf = ix0 + dx
        yf = iy0 + dy
        valid = ((xf >= 0) & (xf <= FW - 1) & (yf >= 0) & (yf <= FH - 1))
        xi = jnp.clip(xf, 0, FW - 1).astype(jnp.int32)
        yi = jnp.clip(yf, 0, FH - 1).astype(jnp.int32)
        idx_refs[k][...] = img * HW + yi * FW + xi
        w_refs[k][...] = wx * wy * valid.astype(jnp.float32)


def _projections(x, Wq, bq2, Woffp, boffp2, Wpv, bpv2):
    bm = 256
    grid = (B // bm,)
    return pl.pallas_call(
        _proj_body,
        grid=grid,
        in_specs=[
            pl.BlockSpec((bm, IN), lambda i: (i, 0)),
            pl.BlockSpec((IN, OUT), lambda i: (0, 0)),
            pl.BlockSpec((1, OUT), lambda i: (0, 0)),
            pl.BlockSpec((IN, 2 * L), lambda i: (0, 0)),
            pl.BlockSpec((1, 2 * L), lambda i: (0, 0)),
            pl.BlockSpec((IN, 4), lambda i: (0, 0)),
            pl.BlockSpec((1, 4), lambda i: (0, 0)),
        ],
        out_specs=[pl.BlockSpec((bm, OUT), lambda i: (i, 0))]
        + [pl.BlockSpec((bm, L), lambda i: (i, 0))] * 8
        + [pl.BlockSpec((bm, 4), lambda i: (i, 0))],
        out_shape=[jax.ShapeDtypeStruct((B, OUT), jnp.float32)]
        + [jax.ShapeDtypeStruct((B, L), jnp.int32)] * 4
        + [jax.ShapeDtypeStruct((B, L), jnp.float32)] * 4
        + [jax.ShapeDtypeStruct((B, 4), jnp.float32)],
    )(x, Wq, bq2, Woffp, boffp2, Wpv, bpv2)


def _sc_gather_combine(table, idxs, ws):
    mesh = plsc.VectorSubcoreMesh(core_axis_name="c", subcore_axis_name="s")

    @functools.partial(
        pl.kernel,
        mesh=mesh,
        out_type=jax.ShapeDtypeStruct((N, P * DK), jnp.float32),
        scratch_types=[
            pltpu.VMEM((4, _BPW, L), jnp.int32),
            pltpu.VMEM((4, _BPW, L), jnp.float32),
            pltpu.VMEM((4 * L, 2 * C), jnp.float32),
            pltpu.VMEM((H, P * DK), jnp.float32),
            pltpu.SemaphoreType.DMA,
            pltpu.SemaphoreType.DMA,
        ],
    )
    def k(table_hbm, i0, i1, i2, i3, w0, w1, w2, w3, out_hbm,
          idx_v, w_v, rows_v, out_v, semA, semB):
        wid = lax.axis_index("s") * 2 + lax.axis_index("c")
        base = wid * _BPW
        for kk, r in enumerate((i0, i1, i2, i3)):
            pltpu.sync_copy(r.at[pl.ds(base, _BPW)], idx_v.at[kk])
        for kk, r in enumerate((w0, w1, w2, w3)):
            pltpu.sync_copy(r.at[pl.ds(base, _BPW)], w_v.at[kk])

        def body(t, _):
            cpsA = [
                pltpu.async_copy(table_hbm.at[idx_v.at[kk, t]],
                                 rows_v.at[pl.ds(kk * L, L)], semA)
                for kk in range(2)
            ]
            cpsB = [
                pltpu.async_copy(table_hbm.at[idx_v.at[kk, t]],
                                 rows_v.at[pl.ds(kk * L, L)], semB)
                for kk in range(2, 4)
            ]
            for cp in cpsA:
                cp.wait()

            def innerA(g, _):
                wv = [w_v[kk, t, pl.ds(g * 16, 16)] for kk in range(2)]
                for j in range(16):
                    l = g * 16 + j
                    ro = g * 2 + j // P
                    co = (j % P) * DK
                    for c4 in range(C // 16):
                        s = pl.ds(c4 * 16, 16)
                        acc = wv[0][j] * rows_v[l, s]
                        acc = acc + wv[1][j] * rows_v[L + l, s]
                        out_v[ro, pl.ds(co + c4 * 16, 16)] = acc
                return 0

            lax.fori_loop(0, L // 16, innerA, 0)
            for cp in cpsB:
                cp.wait()

            def innerB(g, _):
                wv = [w_v[kk, t, pl.ds(g * 16, 16)] for kk in range(2, 4)]
                for j in range(16):
                    l = g * 16 + j
                    ro = g * 2 + j // P
                    co = (j % P) * DK
                    for c4 in range(C // 16):
                        s = pl.ds(c4 * 16, 16)
                        acc = out_v[ro, pl.ds(co + c4 * 16, 16)]
                        acc = acc + wv[0][j] * rows_v[2 * L + l, s]
                        acc = acc + wv[1][j] * rows_v[3 * L + l, s]
                        out_v[ro, pl.ds(co + c4 * 16, 16)] = acc
                return 0

            lax.fori_loop(0, L // 16, innerB, 0)
            pltpu.sync_copy(out_v, out_hbm.at[pl.ds((base + t) * H, H)])
            return 0

        lax.fori_loop(0, _BPW, body, 0)

    return k(table, *idxs, *ws)


def _attn_body(q_ref, s_ref, o_ref):
    q = q_ref[...]
    s = s_ref[...]
    dots = [jnp.sum(q * s[:, p * DK:(p + 1) * DK], axis=1, keepdims=True)
            for p in range(P)]
    scores = jnp.concatenate(dots, axis=1) * (DK ** -0.5)
    m = jnp.max(scores, axis=1, keepdims=True)
    e = jnp.exp(scores - m)
    a = e / jnp.sum(e, axis=1, keepdims=True)
    o = a[:, 0:1] * s[:, :DK]
    for p in range(1, P):
        o = o + a[:, p:p + 1] * s[:, p * DK:(p + 1) * DK]
    o_ref[...] = o


def _attention(q2, s2):
    br = 2048
    return pl.pallas_call(
        _attn_body,
        grid=(N // br,),
        in_specs=[
            pl.BlockSpec((br, DK), lambda i: (i, 0)),
            pl.BlockSpec((br, P * DK), lambda i: (i, 0)),
        ],
        out_specs=pl.BlockSpec((br, DK), lambda i: (i, 0)),
        out_shape=jax.ShapeDtypeStruct((N, DK), jnp.float32),
    )(q2, s2)


def _mm_body(a_ref, w_ref, b_ref, o_ref):
    o_ref[...] = jnp.dot(a_ref[...], w_ref[...],
                         preferred_element_type=jnp.float32) + b_ref[...]


def _out_matmul(a, Wout, bout2):
    bm = 128
    return pl.pallas_call(
        _mm_body,
        grid=(B // bm,),
        in_specs=[
            pl.BlockSpec((bm, OUT), lambda i: (i, 0)),
            pl.BlockSpec((OUT, OUT), lambda i: (0, 0)),
            pl.BlockSpec((1, OUT), lambda i: (0, 0)),
        ],
        out_specs=pl.BlockSpec((bm, OUT), lambda i: (i, 0)),
        out_shape=jax.ShapeDtypeStruct((B, OUT), jnp.float32),
    )(a, Wout, bout2)


@jax.jit
def kernel(x, feature_map, Wq, bq, Woff, boff, Wpos, bpos, Wvel, bvel,
           Wout, bout):
    Woffp = Woff.reshape(IN, L, 2).transpose(0, 2, 1).reshape(IN, 2 * L)
    boffp = boff.reshape(L, 2).transpose(1, 0).reshape(2 * L)
    Wpv = jnp.concatenate([Wpos, Wvel], axis=1)
    bpv = jnp.concatenate([bpos, bvel])

    table = _build_table(feature_map.reshape(B, C, HW)).reshape(B * HW, 2 * C)
    (q, idx0, idx1, idx2, idx3, w0, w1, w2, w3, pv) = _projections(
        x, Wq, bq.reshape(1, OUT), Woffp, boffp.reshape(1, 2 * L),
        Wpv, bpv.reshape(1, 4))
    samp = _sc_gather_combine(table, (idx0, idx1, idx2, idx3),
                              (w0, w1, w2, w3))
    aout = _attention(q.reshape(N, DK), samp)
    out = _out_matmul(aout.reshape(B, OUT), Wout, bout.reshape(1, OUT))
    return out, pv[:, 0:2], pv[:, 2:4]
